# baseline (speedup 1.0000x reference)
import jax
import jax.numpy as jnp
from jax import lax
from jax.experimental import pallas as pl
from jax.experimental.pallas import tpu as pltpu
from jax.experimental.pallas import tpu_sc as plsc

NC = 2
NS = 16
L = 16
NW = NC * NS

D = 64
DW = D // 2
CHUNK = 400
GROUPS = CHUNK // L
NBUF = 2


def _rsqrt16(x):
    xi = lax.bitcast_convert_type(x, jnp.int32)
    yi = jnp.int32(0x5F3759DF) - lax.shift_right_arithmetic(xi, 1)
    y = lax.bitcast_convert_type(yi, jnp.float32)
    for _ in range(3):
        y = y * (1.5 - 0.5 * x * y * y)
    return y


def _sc_body(idx_hbm, table_hbm, out_hbm, idx_all, rows_pk, out_v, in_sem,
             out_sem):
    n_rows = idx_hbm.shape[0]
    per_w = n_rows // NW
    nchunk = per_w // CHUNK

    wid = lax.axis_index("s") * NC + lax.axis_index("c")
    wbase = wid * per_w
    lane = lax.iota(jnp.int32, L)

    pltpu.sync_copy(idx_hbm.at[pl.ds(wbase, per_w)], idx_all)

    def fetch(ii, b):
        base = ii * CHUNK
        for k in range(GROUPS):
            pltpu.async_copy(
                table_hbm.at[idx_all.at[pl.ds(base + k * L, L)]],
                rows_pk.at[b].at[pl.ds(k * L, L)],
                in_sem.at[b],
            )

    def wait_fetch(b):
        pltpu.make_async_copy(
            table_hbm.at[pl.ds(0, CHUNK)],
            rows_pk.at[b],
            in_sem.at[b],
        ).wait()

    def wait_store(ii, b):
        pltpu.make_async_copy(
            out_v.at[b],
            out_hbm.at[pl.ds(wbase + ii * CHUNK, CHUNK)],
            out_sem.at[b],
        ).wait()

    def compute(b):
        packed = rows_pk.at[b]
        outv = out_v.at[b]
        two = jnp.full((L,), 2, jnp.int32)
        m31 = jnp.full((L,), DW - 1, jnp.int32)
        himask = jnp.full((L,), jnp.int32(-65536))

        @pl.loop(0, GROUPS)
        def _group(g):
            rows = g * L + lane
            acc0 = jnp.zeros((L,), jnp.float32)
            acc1 = jnp.zeros((L,), jnp.float32)
            pcs = [(lane + k) & m31 for k in range(2)]
            for j in range(DW):
                k = j % 2
                w = plsc.load_gather(packed, [rows, pcs[k]])
                pcs[k] = (pcs[k] + two) & m31
                flo = lax.bitcast_convert_type(
                    lax.shift_left(w, 16), jnp.float32)
                fhi = lax.bitcast_convert_type(w & himask, jnp.float32)
                acc0 = acc0 + flo * flo
                acc1 = acc1 + fhi * fhi
            tot = acc0 + acc1
            s = jnp.minimum(1.0, _rsqrt16(jnp.maximum(tot, 1e-12)))
            pcs = [(lane + k) & m31 for k in range(2)]
            for j in range(DW):
                k = j % 2
                pc = pcs[k]
                w = plsc.load_gather(packed, [rows, pc])
                pcs[k] = (pc + two) & m31
                flo = lax.bitcast_convert_type(
                    lax.shift_left(w, 16), jnp.float32)
                fhi = lax.bitcast_convert_type(w & himask, jnp.float32)
                ce = pc + pc
                plsc.store_scatter(outv, [rows, ce], flo * s)
                plsc.store_scatter(outv, [rows, ce + 1], fhi * s)

    fetch(0, 0)

    @pl.loop(0, nchunk // NBUF)
    def _pair(i2):
        for b in range(NBUF):
            ii = i2 * NBUF + b
            nxt = ii + 1

            @pl.when(nxt < nchunk)
            def _prefetch():
                @pl.when(nxt > 1)
                def _drain_store():
                    wait_store(ii - 1, 1 - b)

                fetch(nxt, 1 - b)

            wait_fetch(b)
            compute(b)
            pltpu.async_copy(
                out_v.at[b],
                out_hbm.at[pl.ds(wbase + ii * CHUNK, CHUNK)],
                out_sem.at[b],
            )

    for b in range(NBUF):
        wait_store(nchunk - NBUF + b, (nchunk - NBUF + b) % NBUF)


def kernel(xc_padded, table):
    b, s = xc_padded.shape
    n = b * s
    idx = xc_padded.reshape(n)
    u = lax.bitcast_convert_type(table, jnp.uint32)
    r = ((u + 0x7FFF + ((u >> 16) & 1)) >> 16).astype(jnp.int32)
    table_pk = r[:, ::2] | (r[:, 1::2] << 16)

    mesh = plsc.VectorSubcoreMesh(
        core_axis_name="c", subcore_axis_name="s",
        num_cores=NC, num_subcores=NS,
    )
    run = pl.kernel(
        _sc_body,
        out_type=jax.ShapeDtypeStruct((n, D), jnp.float32),
        mesh=mesh,
        scratch_types=[
            pltpu.VMEM((n // NW,), jnp.int32),
            pltpu.VMEM((NBUF, CHUNK, DW), jnp.int32),
            pltpu.VMEM((NBUF, CHUNK, D), jnp.float32),
            pltpu.SemaphoreType.DMA((NBUF,)),
            pltpu.SemaphoreType.DMA((NBUF,)),
        ],
        compiler_params=pltpu.CompilerParams(
            needs_layout_passes=False, use_tc_tiling_on_sc=False
        ),
    )
    out = run(idx, table_pk)
    return out.reshape(b, s, D)


# speedup vs baseline: 4.6840x; 4.6840x over previous
import jax
import jax.numpy as jnp
from jax import lax
from jax.experimental import pallas as pl
from jax.experimental.pallas import tpu as pltpu
from jax.experimental.pallas import tpu_sc as plsc

NC = 2
NS = 16
L = 16
NW = NC * NS

D = 64
DW = D // 2
CHUNK = 400
GROUPS = CHUNK // L
NBUF = 2


def _rsqrt16(x):
    xi = lax.bitcast_convert_type(x, jnp.int32)
    yi = jnp.int32(0x5F3759DF) - lax.shift_right_arithmetic(xi, 1)
    y = lax.bitcast_convert_type(yi, jnp.float32)
    for _ in range(3):
        y = y * (1.5 - 0.5 * x * y * y)
    return y


def _sc_body(idx_hbm, table_hbm, out_hbm, idx_all, rows_pk, out_v, in_sem,
             out_sem):
    n_rows = idx_hbm.shape[0]
    per_w = n_rows // NW
    nchunk = per_w // CHUNK

    wid = lax.axis_index("s") * NC + lax.axis_index("c")
    wbase = wid * per_w
    lane = lax.iota(jnp.int32, L)

    pltpu.sync_copy(idx_hbm.at[pl.ds(wbase, per_w)], idx_all)

    def fetch(ii, b):
        base = ii * CHUNK
        for k in range(GROUPS):
            pltpu.async_copy(
                table_hbm.at[idx_all.at[pl.ds(base + k * L, L)]],
                rows_pk.at[b].at[pl.ds(k * L, L)],
                in_sem.at[b],
            )

    def wait_fetch(b):
        pltpu.make_async_copy(
            table_hbm.at[pl.ds(0, CHUNK)],
            rows_pk.at[b],
            in_sem.at[b],
        ).wait()

    def wait_store(ii, b):
        pltpu.make_async_copy(
            out_v.at[b],
            out_hbm.at[pl.ds(wbase + ii * CHUNK, CHUNK)],
            out_sem.at[b],
        ).wait()

    def compute(b):
        packed = rows_pk.at[b]
        outv = out_v.at[b]
        two = jnp.full((L,), 2, jnp.int32)
        m31 = jnp.full((L,), DW - 1, jnp.int32)
        himask = jnp.full((L,), jnp.int32(-65536))

        @pl.loop(0, GROUPS)
        def _group(g):
            rows = g * L + lane
            acc0 = jnp.zeros((L,), jnp.float32)
            acc1 = jnp.zeros((L,), jnp.float32)
            pcs = [(lane + k) & m31 for k in range(2)]
            for j in range(DW):
                k = j % 2
                w = plsc.load_gather(packed, [rows, pcs[k]])
                pcs[k] = (pcs[k] + two) & m31
                flo = lax.bitcast_convert_type(
                    lax.shift_left(w, 16), jnp.float32)
                fhi = lax.bitcast_convert_type(w & himask, jnp.float32)
                acc0 = acc0 + flo * flo
                acc1 = acc1 + fhi * fhi
            tot = acc0 + acc1
            s = jnp.minimum(1.0, _rsqrt16(jnp.maximum(tot, 1e-12)))
            pcs = [(lane + k) & m31 for k in range(2)]
            for j in range(DW):
                k = j % 2
                pc = pcs[k]
                w = plsc.load_gather(packed, [rows, pc])
                pcs[k] = (pc + two) & m31
                flo = lax.bitcast_convert_type(
                    lax.shift_left(w, 16), jnp.float32)
                fhi = lax.bitcast_convert_type(w & himask, jnp.float32)
                plsc.store_scatter(outv, [rows, pc], flo * s)
                plsc.store_scatter(outv, [rows, pc + DW], fhi * s)

    fetch(0, 0)

    @pl.loop(0, nchunk // NBUF)
    def _pair(i2):
        for b in range(NBUF):
            ii = i2 * NBUF + b
            nxt = ii + 1

            @pl.when(nxt < nchunk)
            def _prefetch():
                @pl.when(nxt > 1)
                def _drain_store():
                    wait_store(ii - 1, 1 - b)

                fetch(nxt, 1 - b)

            wait_fetch(b)
            compute(b)
            pltpu.async_copy(
                out_v.at[b],
                out_hbm.at[pl.ds(wbase + ii * CHUNK, CHUNK)],
                out_sem.at[b],
            )

    for b in range(NBUF):
        wait_store(nchunk - NBUF + b, (nchunk - NBUF + b) % NBUF)


def kernel(xc_padded, table):
    b, s = xc_padded.shape
    n = b * s
    idx = xc_padded.reshape(n)
    u = lax.bitcast_convert_type(table, jnp.uint32)
    r = ((u + 0x7FFF + ((u >> 16) & 1)) >> 16).astype(jnp.int32)
    table_pk = r[:, :DW] | (r[:, DW:] << 16)

    mesh = plsc.VectorSubcoreMesh(
        core_axis_name="c", subcore_axis_name="s",
        num_cores=NC, num_subcores=NS,
    )
    run = pl.kernel(
        _sc_body,
        out_type=jax.ShapeDtypeStruct((n, D), jnp.float32),
        mesh=mesh,
        scratch_types=[
            pltpu.VMEM((n // NW,), jnp.int32),
            pltpu.VMEM((NBUF, CHUNK, DW), jnp.int32),
            pltpu.VMEM((NBUF, CHUNK, D), jnp.float32),
            pltpu.SemaphoreType.DMA((NBUF,)),
            pltpu.SemaphoreType.DMA((NBUF,)),
        ],
        compiler_params=pltpu.CompilerParams(
            needs_layout_passes=False, use_tc_tiling_on_sc=False
        ),
    )
    out = run(idx, table_pk)
    return out.reshape(b, s, D)


# speedup vs baseline: 4.6904x; 1.0014x over previous
import jax
import jax.numpy as jnp
from jax import lax
from jax.experimental import pallas as pl
from jax.experimental.pallas import tpu as pltpu
from jax.experimental.pallas import tpu_sc as plsc

NC = 2
NS = 16
L = 16
NW = NC * NS

D = 64
DW = D // 2
CHUNK = 400
GROUPS = CHUNK // L
NBUF = 2


def _rsqrt16(x):
    xi = lax.bitcast_convert_type(x, jnp.int32)
    yi = jnp.int32(0x5F3759DF) - lax.shift_right_arithmetic(xi, 1)
    y = lax.bitcast_convert_type(yi, jnp.float32)
    for _ in range(3):
        y = y * (1.5 - 0.5 * x * y * y)
    return y


def _sc_body(idx_hbm, table_hbm, out_hbm, idx_all, rows_pk, out_v, in_sem,
             out_sem):
    n_rows = idx_hbm.shape[0]
    per_w = n_rows // NW
    nchunk = per_w // CHUNK

    wid = lax.axis_index("s") * NC + lax.axis_index("c")
    wbase = wid * per_w
    lane = lax.iota(jnp.int32, L)

    pltpu.sync_copy(idx_hbm.at[pl.ds(wbase, per_w)], idx_all)

    def fetch(ii, b):
        base = ii * CHUNK
        for k in range(GROUPS):
            pltpu.async_copy(
                table_hbm.at[idx_all.at[pl.ds(base + k * L, L)]],
                rows_pk.at[b].at[pl.ds(k * L, L)],
                in_sem.at[b],
            )

    def wait_fetch(b):
        pltpu.make_async_copy(
            table_hbm.at[pl.ds(0, CHUNK)],
            rows_pk.at[b],
            in_sem.at[b],
        ).wait()

    def wait_store(ii, b):
        pltpu.make_async_copy(
            out_v.at[b],
            out_hbm.at[pl.ds(wbase + ii * CHUNK, CHUNK)],
            out_sem.at[b],
        ).wait()

    def compute(b):
        packed = rows_pk.at[b]
        outv = out_v.at[b]
        two = jnp.full((L,), 2, jnp.int32)
        m31 = jnp.full((L,), DW - 1, jnp.int32)
        himask = jnp.full((L,), jnp.int32(-65536))

        @pl.loop(0, GROUPS)
        def _group(g):
            rows = g * L + lane
            acc0 = jnp.zeros((L,), jnp.float32)
            acc1 = jnp.zeros((L,), jnp.float32)
            pcs = [(lane + k) & m31 for k in range(2)]
            for j in range(DW):
                k = j % 2
                w = plsc.load_gather(packed, [rows, pcs[k]])
                pcs[k] = (pcs[k] + two) & m31
                flo = lax.bitcast_convert_type(
                    lax.shift_left(w, 16), jnp.float32)
                fhi = lax.bitcast_convert_type(w & himask, jnp.float32)
                acc0 = acc0 + flo * flo
                acc1 = acc1 + fhi * fhi
            tot = acc0 + acc1
            s = jnp.minimum(1.0, _rsqrt16(jnp.maximum(tot, 1e-12)))
            pcs = [(lane + k) & m31 for k in range(2)]
            for j in range(DW):
                k = j % 2
                pc = pcs[k]
                w = plsc.load_gather(packed, [rows, pc])
                pcs[k] = (pc + two) & m31
                flo = lax.bitcast_convert_type(
                    lax.shift_left(w, 16), jnp.float32)
                fhi = lax.bitcast_convert_type(w & himask, jnp.float32)
                plsc.store_scatter(outv, [rows, pc], flo * s)
                plsc.store_scatter(outv, [rows, pc + DW], fhi * s)

    fetch(0, 0)

    @pl.loop(0, nchunk // NBUF)
    def _pair(i2):
        for b in range(NBUF):
            ii = i2 * NBUF + b
            nxt = ii + 1

            @pl.when(nxt < nchunk)
            def _prefetch():
                @pl.when(nxt > 1)
                def _drain_store():
                    wait_store(ii - 1, 1 - b)

                fetch(nxt, 1 - b)

            wait_fetch(b)
            compute(b)
            pltpu.async_copy(
                out_v.at[b],
                out_hbm.at[pl.ds(wbase + ii * CHUNK, CHUNK)],
                out_sem.at[b],
            )

    for b in range(NBUF):
        wait_store(nchunk - NBUF + b, (nchunk - NBUF + b) % NBUF)


def kernel(xc_padded, table):
    b, s = xc_padded.shape
    n = b * s
    idx = xc_padded.reshape(n)
    u = lax.bitcast_convert_type(table, jnp.uint32)
    r = ((u + 0x7FFF + ((u >> 16) & 1)) >> 16).astype(jnp.int32)
    table_pk = r[:, :DW] | (r[:, DW:] << 16)
    table_pk = lax.optimization_barrier(table_pk)

    mesh = plsc.VectorSubcoreMesh(
        core_axis_name="c", subcore_axis_name="s",
        num_cores=NC, num_subcores=NS,
    )
    run = pl.kernel(
        _sc_body,
        out_type=jax.ShapeDtypeStruct((n, D), jnp.float32),
        mesh=mesh,
        scratch_types=[
            pltpu.VMEM((n // NW,), jnp.int32),
            pltpu.VMEM((NBUF, CHUNK, DW), jnp.int32),
            pltpu.VMEM((NBUF, CHUNK, D), jnp.float32),
            pltpu.SemaphoreType.DMA((NBUF,)),
            pltpu.SemaphoreType.DMA((NBUF,)),
        ],
        compiler_params=pltpu.CompilerParams(
            needs_layout_passes=False, use_tc_tiling_on_sc=False
        ),
    )
    out = run(idx, table_pk)
    return out.reshape(b, s, D)


# speedup vs baseline: 7.7066x; 1.6431x over previous
import jax
import jax.numpy as jnp
from jax import lax
from jax.experimental import pallas as pl
from jax.experimental.pallas import tpu as pltpu
from jax.experimental.pallas import tpu_sc as plsc

NC = 2
NS = 16
L = 16
NW = NC * NS

D = 64
CHUNK = 512
DMA_SPLIT = 4
SUB = CHUNK // DMA_SPLIT
GROUPS = CHUNK // L
NBUF = 2


def _rsqrt16(x):
    xi = lax.bitcast_convert_type(x, jnp.int32)
    yi = jnp.int32(0x5F3759DF) - lax.shift_right_arithmetic(xi, 1)
    y = lax.bitcast_convert_type(yi, jnp.float32)
    for _ in range(3):
        y = y * (1.5 - 0.5 * x * y * y)
    return y


def _sc_body(idx_hbm, table_hbm, out_hbm, idx_all, rows_v, in_sem, out_sem):
    n_rows = idx_hbm.shape[0]
    per_w = n_rows // NW
    nchunk = per_w // CHUNK

    wid = lax.axis_index("s") * NC + lax.axis_index("c")
    wbase = wid * per_w
    lane = lax.iota(jnp.int32, L)

    pltpu.sync_copy(idx_hbm.at[pl.ds(wbase, per_w)], idx_all)

    def fetch(ii, b):
        base = ii * CHUNK
        for k in range(DMA_SPLIT):
            pltpu.async_copy(
                table_hbm.at[idx_all.at[pl.ds(base + k * SUB, SUB)]],
                rows_v.at[b].at[pl.ds(k * SUB, SUB)],
                in_sem.at[b],
            )

    def wait_fetch(ii, b):
        pltpu.make_async_copy(
            out_hbm.at[pl.ds(wbase + ii * CHUNK, CHUNK)],
            rows_v.at[b],
            in_sem.at[b],
        ).wait()

    def wait_store(ii, b):
        pltpu.make_async_copy(
            rows_v.at[b],
            out_hbm.at[pl.ds(wbase + ii * CHUNK, CHUNK)],
            out_sem.at[b],
        ).wait()

    def compute(b):
        ref = rows_v.at[b]
        four = jnp.full((L,), 4, jnp.int32)
        m63 = jnp.full((L,), D - 1, jnp.int32)

        @pl.loop(0, GROUPS)
        def _group(g):
            rows = g * L + lane
            accs = [jnp.zeros((L,), jnp.float32) for _ in range(4)]
            cs = [(lane + k) & m63 for k in range(4)]
            for j in range(D):
                k = j % 4
                v = plsc.load_gather(ref, [rows, cs[k]])
                accs[k] = accs[k] + v * v
                cs[k] = (cs[k] + four) & m63
            tot = (accs[0] + accs[1]) + (accs[2] + accs[3])
            s = jnp.minimum(1.0, _rsqrt16(jnp.maximum(tot, 1e-12)))
            cs = [(lane + k) & m63 for k in range(4)]
            for j0 in range(0, D, 4):
                vals = []
                cols = []
                for k in range(4):
                    c = cs[k]
                    vals.append(plsc.load_gather(ref, [rows, c]))
                    cols.append(c)
                    cs[k] = (c + four) & m63
                for k in range(4):
                    plsc.store_scatter(ref, [rows, cols[k]], vals[k] * s)

    fetch(0, 0)

    @pl.loop(0, nchunk // NBUF)
    def _pair(i2):
        for b in range(NBUF):
            ii = i2 * NBUF + b
            nxt = ii + 1

            @pl.when(nxt < nchunk)
            def _prefetch():
                @pl.when(nxt > 1)
                def _drain_store():
                    wait_store(ii - 1, 1 - b)

                fetch(nxt, 1 - b)

            wait_fetch(ii, b)
            compute(b)
            pltpu.async_copy(
                rows_v.at[b],
                out_hbm.at[pl.ds(wbase + ii * CHUNK, CHUNK)],
                out_sem.at[b],
            )

    for b in range(NBUF):
        wait_store(nchunk - NBUF + b, b)


def kernel(xc_padded, table):
    b, s = xc_padded.shape
    n = b * s
    idx = xc_padded.reshape(n)

    mesh = plsc.VectorSubcoreMesh(
        core_axis_name="c", subcore_axis_name="s",
        num_cores=NC, num_subcores=NS,
    )
    run = pl.kernel(
        _sc_body,
        out_type=jax.ShapeDtypeStruct((n, D), jnp.float32),
        mesh=mesh,
        scratch_types=[
            pltpu.VMEM((n // NW,), jnp.int32),
            pltpu.VMEM((NBUF, CHUNK, D), jnp.float32),
            pltpu.SemaphoreType.DMA((NBUF,)),
            pltpu.SemaphoreType.DMA((NBUF,)),
        ],
        compiler_params=pltpu.CompilerParams(
            needs_layout_passes=False, use_tc_tiling_on_sc=False
        ),
    )
    out = run(idx, table)
    return out.reshape(b, s, D)


# speedup vs baseline: 8.0669x; 1.0467x over previous
import jax
import jax.numpy as jnp
from jax import lax
from jax.experimental import pallas as pl
from jax.experimental.pallas import tpu as pltpu
from jax.experimental.pallas import tpu_sc as plsc

NC = 2
NS = 16
L = 16
NW = NC * NS

D = 64
CHUNK = 512
DMA_SPLIT = 4
SUB = CHUNK // DMA_SPLIT
GROUPS = CHUNK // L
NBUF = 3


def _rsqrt16(x):
    xi = lax.bitcast_convert_type(x, jnp.int32)
    yi = jnp.int32(0x5F3759DF) - lax.shift_right_arithmetic(xi, 1)
    y = lax.bitcast_convert_type(yi, jnp.float32)
    for _ in range(3):
        y = y * (1.5 - 0.5 * x * y * y)
    return y


def _sc_body(idx_hbm, table_hbm, out_hbm, idx_all, rows_v, in_sem, out_sem):
    n_rows = idx_hbm.shape[0]
    per_w = n_rows // NW
    nchunk = per_w // CHUNK

    wid = lax.axis_index("s") * NC + lax.axis_index("c")
    wbase = wid * per_w
    lane = lax.iota(jnp.int32, L)

    pltpu.sync_copy(idx_hbm.at[pl.ds(wbase, per_w)], idx_all)

    def fetch(ii, b):
        base = ii * CHUNK
        for k in range(DMA_SPLIT):
            pltpu.async_copy(
                table_hbm.at[idx_all.at[pl.ds(base + k * SUB, SUB)]],
                rows_v.at[b].at[pl.ds(k * SUB, SUB)],
                in_sem.at[b],
            )

    def wait_fetch(ii, b):
        pltpu.make_async_copy(
            out_hbm.at[pl.ds(wbase + ii * CHUNK, CHUNK)],
            rows_v.at[b],
            in_sem.at[b],
        ).wait()

    def wait_store(ii, b):
        pltpu.make_async_copy(
            rows_v.at[b],
            out_hbm.at[pl.ds(wbase + ii * CHUNK, CHUNK)],
            out_sem.at[b],
        ).wait()

    def compute(b):
        ref = rows_v.at[b]
        four = jnp.full((L,), 4, jnp.int32)
        m63 = jnp.full((L,), D - 1, jnp.int32)

        @pl.loop(0, GROUPS)
        def _group(g):
            rows = g * L + lane
            accs = [jnp.zeros((L,), jnp.float32) for _ in range(4)]
            cs = [(lane + k) & m63 for k in range(4)]
            for j in range(D):
                k = j % 4
                v = plsc.load_gather(ref, [rows, cs[k]])
                accs[k] = accs[k] + v * v
                cs[k] = (cs[k] + four) & m63
            tot = (accs[0] + accs[1]) + (accs[2] + accs[3])
            s = jnp.minimum(1.0, _rsqrt16(jnp.maximum(tot, 1e-12)))
            cs = [(lane + k) & m63 for k in range(4)]
            for j0 in range(0, D, 4):
                vals = []
                cols = []
                for k in range(4):
                    c = cs[k]
                    vals.append(plsc.load_gather(ref, [rows, c]))
                    cols.append(c)
                    cs[k] = (c + four) & m63
                for k in range(4):
                    plsc.store_scatter(ref, [rows, cols[k]], vals[k] * s)

    fetch(0, 0)
    outer = (nchunk + NBUF - 1) // NBUF

    @pl.loop(0, outer)
    def _ring(i2):
        for b in range(NBUF):
            ii = i2 * NBUF + b

            @pl.when(ii < nchunk)
            def _chunk():
                nxt = ii + 1
                nb = (b + 1) % NBUF

                @pl.when(nxt < nchunk)
                def _prefetch():
                    @pl.when(nxt > NBUF - 1)
                    def _drain_store():
                        wait_store(nxt - NBUF, nb)

                    fetch(nxt, nb)

                wait_fetch(ii, b)
                compute(b)
                pltpu.async_copy(
                    rows_v.at[b],
                    out_hbm.at[pl.ds(wbase + ii * CHUNK, CHUNK)],
                    out_sem.at[b],
                )

    for k in range(NBUF):
        c = nchunk - NBUF + k
        wait_store(c, c % NBUF)


def kernel(xc_padded, table):
    b, s = xc_padded.shape
    n = b * s
    idx = xc_padded.reshape(n)

    mesh = plsc.VectorSubcoreMesh(
        core_axis_name="c", subcore_axis_name="s",
        num_cores=NC, num_subcores=NS,
    )
    run = pl.kernel(
        _sc_body,
        out_type=jax.ShapeDtypeStruct((n, D), jnp.float32),
        mesh=mesh,
        scratch_types=[
            pltpu.VMEM((n // NW,), jnp.int32),
            pltpu.VMEM((NBUF, CHUNK, D), jnp.float32),
            pltpu.SemaphoreType.DMA((NBUF,)),
            pltpu.SemaphoreType.DMA((NBUF,)),
        ],
        compiler_params=pltpu.CompilerParams(
            needs_layout_passes=False, use_tc_tiling_on_sc=False
        ),
    )
    out = run(idx, table)
    return out.reshape(b, s, D)
